# baseline (speedup 1.0000x reference)
import functools

import jax
import jax.numpy as jnp
from jax import lax
from jax.experimental import pallas as pl
from jax.experimental.pallas import tpu as pltpu
from jax.experimental.pallas import tpu_sc as plsc

B = 4096
L = 128
F = 128
V = 100000

VBLK = 8192
VPAD = ((V + VBLK - 1) // VBLK) * VBLK

NC = 2
NS = 16
NW = NC * NS
N_IDX = B * L
PER_W = N_IDX // NW
CHUNK = 4096
NCHUNK = PER_W // CHUNK
NBUF = 2
LANES = 16


def _rowmean_body(emb_ref, s_ref):
    e3 = emb_ref[...].reshape(VBLK // 128, 128, F)
    ones = jnp.full((F,), 1.0 / F, dtype=jnp.float32)
    m = lax.dot_general(e3, ones, (((2,), (0,)), ((), ())),
                        preferred_element_type=jnp.float32)
    pad0 = (pl.program_id(0) == 0) & (
        (lax.broadcasted_iota(jnp.int32, m.shape, 0)
         + lax.broadcasted_iota(jnp.int32, m.shape, 1)) == 0)
    s_ref[...] = jnp.where(pad0, 0.0, m).astype(jnp.bfloat16)


def _row_means(emb):
    sb = pl.pallas_call(
        _rowmean_body,
        grid=(VPAD // VBLK,),
        in_specs=[pl.BlockSpec((VBLK, F), lambda i: (i, 0))],
        out_specs=pl.BlockSpec((VBLK // 128, 128), lambda i: (i, 0)),
        out_shape=jax.ShapeDtypeStruct((VPAD // 128, 128), jnp.bfloat16),
    )(emb)
    return lax.bitcast_convert_type(sb.reshape(VPAD // 2, 2), jnp.int32)


@functools.cache
def _sc_gather_fn():
    mesh = plsc.VectorSubcoreMesh(
        core_axis_name="c", subcore_axis_name="s",
        num_cores=NC, num_subcores=NS)

    @functools.partial(
        pl.kernel,
        mesh=mesh,
        out_type=jax.ShapeDtypeStruct((N_IDX,), jnp.float32),
        compiler_params=pltpu.CompilerParams(needs_layout_passes=False),
        scratch_types=[
            pltpu.VMEM((VPAD // 2,), jnp.int32),
            pltpu.VMEM((NBUF, CHUNK), jnp.int32),
            pltpu.VMEM((NBUF, CHUNK), jnp.float32),
            pltpu.SemaphoreType.DMA,
            pltpu.SemaphoreType.DMA,
            pltpu.SemaphoreType.DMA,
            pltpu.SemaphoreType.DMA,
            pltpu.SemaphoreType.DMA,
        ],
    )
    def _sc_gather(s_hbm, x_hbm, out_hbm, s_v, idx_v, out_v,
                   s_sem, i_sem0, i_sem1, o_sem0, o_sem1):
        wid = lax.axis_index("s") * NC + lax.axis_index("c")
        base = wid * PER_W
        i_sems = (i_sem0, i_sem1)
        o_sems = (o_sem0, o_sem1)
        s_cp = pltpu.async_copy(s_hbm, s_v, s_sem)
        idx_cps = [
            pltpu.async_copy(x_hbm.at[pl.ds(base + c * CHUNK, CHUNK)],
                             idx_v.at[c], i_sems[c])
            for c in range(NBUF)
        ]
        out_cps = [None] * NBUF
        s_cp.wait()
        for c in range(NCHUNK):
            b = c % NBUF
            idx_cps[b].wait()
            if c >= NBUF:
                out_cps[b].wait()

            @plsc.parallel_loop(0, CHUNK // LANES, unroll=16)
            def _(i):
                off = i * LANES
                idx16 = idx_v[b, pl.ds(off, LANES)]
                w = plsc.load_gather(
                    s_v, [lax.shift_right_logical(idx16, 1)])
                odd = (idx16 & 1) == 1
                lo = lax.shift_left(w, 16)
                hi = w & jnp.int32(-65536)
                out_v[b, pl.ds(off, LANES)] = plsc.bitcast(
                    jnp.where(odd, hi, lo), jnp.float32)

            out_cps[b] = pltpu.async_copy(
                out_v.at[b], out_hbm.at[pl.ds(base + c * CHUNK, CHUNK)],
                o_sems[b])
            if c + NBUF < NCHUNK:
                idx_cps[b] = pltpu.async_copy(
                    x_hbm.at[pl.ds(base + (c + NBUF) * CHUNK, CHUNK)],
                    idx_v.at[b], i_sems[b])
        for c in range(max(0, NCHUNK - NBUF), NCHUNK):
            out_cps[c % NBUF].wait()

    return _sc_gather


def _head_body(p_ref, w_ref, b_ref, g_ref, be_ref, o_ref):
    p = p_ref[...]
    y = lax.dot_general(p, w_ref[...], (((1,), (1,)), ((), ())),
                        preferred_element_type=jnp.float32)
    y = y + b_ref[...]
    mu = jnp.mean(y, axis=0, keepdims=True)
    var = jnp.mean((y - mu) ** 2, axis=0, keepdims=True)
    y = (y - mu) / jnp.sqrt(var + 1e-5) * g_ref[...] + be_ref[...]
    mu2 = jnp.mean(y, axis=1, keepdims=True)
    var2 = jnp.mean((y - mu2) ** 2, axis=1, keepdims=True)
    o_ref[...] = (y - mu2) / jnp.sqrt(var2 + 1e-5)


def _head(pooled, W, b, gamma, beta):
    return pl.pallas_call(
        _head_body,
        out_shape=jax.ShapeDtypeStruct((B, F), jnp.float32),
    )(pooled, W, b.reshape(1, F), gamma.reshape(1, F), beta.reshape(1, F))


def kernel(x, emb, W, b, gamma, beta):
    s = _row_means(emb)
    pooled = _sc_gather_fn()(s, x.reshape(-1))
    return _head(pooled.reshape(B, L), W, b, gamma, beta)


# speedup vs baseline: 1.7737x; 1.7737x over previous
import functools

import jax
import jax.numpy as jnp
from jax import lax
from jax.experimental import pallas as pl
from jax.experimental.pallas import tpu as pltpu
from jax.experimental.pallas import tpu_sc as plsc

B = 4096
L = 128
F = 128
V = 100000

VBLK = 4096
VPAD = 2 * 13 * VBLK
HALF = VPAD // 2

NC = 2
NS = 16
NW = NC * NS
N_IDX = B * L
PER_W = N_IDX // NW
CHUNK = 8192
NCHUNK = PER_W // CHUNK
NBUF = 2
LANES = 16


def _bf16_bits(m):
    bits = lax.bitcast_convert_type(m, jnp.int32)
    return bits + jnp.int32(0x7FFF) + (lax.shift_right_logical(bits, 16) & 1)


def _rowmean_body(lo_ref, hi_ref, s_ref):
    ones = jnp.full((F,), 1.0 / F, dtype=jnp.float32)

    def rmean(ref):
        e3 = ref[...].reshape(VBLK // 128, 128, F)
        return lax.dot_general(e3, ones, (((2,), (0,)), ((), ())),
                               preferred_element_type=jnp.float32)

    m_lo = rmean(lo_ref)
    m_hi = rmean(hi_ref)
    pad0 = (pl.program_id(0) == 0) & (
        (lax.broadcasted_iota(jnp.int32, m_lo.shape, 0)
         + lax.broadcasted_iota(jnp.int32, m_lo.shape, 1)) == 0)
    m_lo = jnp.where(pad0, 0.0, m_lo)
    lo16 = lax.shift_right_logical(_bf16_bits(m_lo), 16)
    hi16 = _bf16_bits(m_hi) & jnp.int32(-65536)
    s_ref[...] = lo16 | hi16


def _row_means(emb):
    nblk = HALF // VBLK
    last = (V - 1) // VBLK
    return pl.pallas_call(
        _rowmean_body,
        grid=(nblk,),
        in_specs=[pl.BlockSpec((VBLK, F), lambda i: (i, 0)),
                  pl.BlockSpec((VBLK, F),
                               lambda i: (jnp.minimum(i + nblk, last), 0))],
        out_specs=pl.BlockSpec((VBLK // 128, 128), lambda i: (i, 0)),
        out_shape=jax.ShapeDtypeStruct((HALF // 128, 128), jnp.int32),
    )(emb, emb).reshape(HALF)


@functools.cache
def _sc_gather_fn():
    mesh = plsc.VectorSubcoreMesh(
        core_axis_name="c", subcore_axis_name="s",
        num_cores=NC, num_subcores=NS)

    @functools.partial(
        pl.kernel,
        mesh=mesh,
        out_type=jax.ShapeDtypeStruct((N_IDX,), jnp.float32),
        compiler_params=pltpu.CompilerParams(needs_layout_passes=False),
        scratch_types=[
            pltpu.VMEM((HALF,), jnp.int32),
            pltpu.VMEM((NBUF, CHUNK), jnp.int32),
            pltpu.VMEM((NBUF, CHUNK), jnp.float32),
            pltpu.SemaphoreType.DMA,
            pltpu.SemaphoreType.DMA,
            pltpu.SemaphoreType.DMA,
            pltpu.SemaphoreType.DMA,
            pltpu.SemaphoreType.DMA,
        ],
    )
    def _sc_gather(s_hbm, x_hbm, out_hbm, s_v, idx_v, out_v,
                   s_sem, i_sem0, i_sem1, o_sem0, o_sem1):
        wid = lax.axis_index("s") * NC + lax.axis_index("c")
        base = wid * PER_W
        i_sems = (i_sem0, i_sem1)
        o_sems = (o_sem0, o_sem1)
        s_cp = pltpu.async_copy(s_hbm, s_v, s_sem)
        idx_cps = [
            pltpu.async_copy(x_hbm.at[pl.ds(base + c * CHUNK, CHUNK)],
                             idx_v.at[c], i_sems[c])
            for c in range(NBUF)
        ]
        out_cps = [None] * NBUF
        s_cp.wait()
        for c in range(NCHUNK):
            b = c % NBUF
            idx_cps[b].wait()
            if c >= NBUF:
                out_cps[b].wait()

            @plsc.parallel_loop(0, CHUNK // LANES, unroll=16)
            def _(i):
                off = i * LANES
                idx16 = idx_v[b, pl.ds(off, LANES)]
                ge = idx16 >= HALF
                wi = idx16 - jnp.where(ge, HALF, 0)
                w = plsc.load_gather(s_v, [wi])
                lo = lax.shift_left(w, 16)
                hi = w & jnp.int32(-65536)
                out_v[b, pl.ds(off, LANES)] = plsc.bitcast(
                    jnp.where(ge, hi, lo), jnp.float32)

            out_cps[b] = pltpu.async_copy(
                out_v.at[b], out_hbm.at[pl.ds(base + c * CHUNK, CHUNK)],
                o_sems[b])
            if c + NBUF < NCHUNK:
                idx_cps[b] = pltpu.async_copy(
                    x_hbm.at[pl.ds(base + (c + NBUF) * CHUNK, CHUNK)],
                    idx_v.at[b], i_sems[b])
        for c in range(max(0, NCHUNK - NBUF), NCHUNK):
            out_cps[c % NBUF].wait()

    return _sc_gather


def _head_body(p_ref, w_ref, b_ref, g_ref, be_ref, o_ref):
    p = p_ref[...]
    y = lax.dot_general(p, w_ref[...], (((1,), (1,)), ((), ())),
                        preferred_element_type=jnp.float32)
    y = y + b_ref[...]
    mu = jnp.mean(y, axis=0, keepdims=True)
    var = jnp.mean((y - mu) ** 2, axis=0, keepdims=True)
    y = (y - mu) / jnp.sqrt(var + 1e-5) * g_ref[...] + be_ref[...]
    mu2 = jnp.mean(y, axis=1, keepdims=True)
    var2 = jnp.mean((y - mu2) ** 2, axis=1, keepdims=True)
    o_ref[...] = (y - mu2) / jnp.sqrt(var2 + 1e-5)


def _head(pooled, W, b, gamma, beta):
    return pl.pallas_call(
        _head_body,
        out_shape=jax.ShapeDtypeStruct((B, F), jnp.float32),
    )(pooled, W, b.reshape(1, F), gamma.reshape(1, F), beta.reshape(1, F))


def kernel(x, emb, W, b, gamma, beta):
    s = _row_means(emb)
    pooled = _sc_gather_fn()(s, x.reshape(-1))
    return _head(pooled.reshape(B, L), W, b, gamma, beta)
